# Initial kernel scaffold; baseline (speedup 1.0000x reference)
import jax
import jax.numpy as jnp
from jax.experimental import pallas as pl


def kernel(user, pos, neg, laplacian_indices, laplacian_values, emb_weight):
    raise NotImplementedError("write your pallas kernel here")


# speedup vs baseline: 3.9709x; 3.9709x over previous
import functools

import jax
import jax.numpy as jnp
from jax import lax
from jax.experimental import pallas as pl
from jax.experimental.pallas import tpu as pltpu
from jax.experimental.pallas import tpu_sc as plsc

NC = 2
NS = 16
NWK = NC * NS
LANES = 16
K = 128


def _spmm_sc(emb, row, col, val, zeros):
    n, h = emb.shape
    e = row.shape[0]
    n_chunks = e // K
    iters = (n_chunks + NWK - 1) // NWK
    rows_per_sub = n // NS
    groups = h // LANES
    mesh = plsc.VectorSubcoreMesh(core_axis_name="c", subcore_axis_name="s")

    @functools.partial(
        pl.kernel,
        out_type=jax.ShapeDtypeStruct((NC, n, h), jnp.float32),
        mesh=mesh,
        compiler_params=pltpu.CompilerParams(needs_layout_passes=False),
        scratch_types=[
            pltpu.VMEM((K,), jnp.int32),
            pltpu.VMEM((K,), jnp.int32),
            pltpu.VMEM((K,), jnp.float32),
            pltpu.VMEM((K, h), jnp.float32),
            pltpu.VMEM_SHARED((n, h), jnp.float32),
            pltpu.SemaphoreType.DMA,
        ],
    )
    def spmm(row_hbm, col_hbm, val_hbm, emb_hbm, zeros_hbm, out_hbm,
             row_v, col_v, val_v, rows_v, acc, sem):
        c = lax.axis_index("c")
        s = lax.axis_index("s")
        wid = s * NC + c
        sub = rows_per_sub // 8 * 8
        tail = n - NS * sub
        sub_lo = s * sub
        pltpu.sync_copy(zeros_hbm.at[pl.ds(sub_lo, sub)],
                        acc.at[pl.ds(sub_lo, sub)])
        if tail:
            @pl.when(s == NS - 1)
            def _():
                pltpu.sync_copy(zeros_hbm.at[pl.ds(NS * sub, tail)],
                                acc.at[pl.ds(NS * sub, tail)])
        plsc.subcore_barrier()

        def chunk_body(i, carry):
            cid = wid + i * NWK

            @pl.when(cid < n_chunks)
            def _():
                base = cid * K
                pltpu.sync_copy(row_hbm.at[pl.ds(base, K)], row_v)
                pltpu.sync_copy(col_hbm.at[pl.ds(base, K)], col_v)
                pltpu.sync_copy(val_hbm.at[pl.ds(base, K)], val_v)
                pltpu.async_copy(emb_hbm.at[col_v], rows_v, sem).wait()

                def scale_row(r, carry2):
                    vb = plsc.load_gather(val_v, [jnp.full((LANES,), r, jnp.int32)])
                    for g in range(groups):
                        sl = pl.ds(g * LANES, LANES)
                        rows_v[r, sl] = rows_v[r, sl] * vb
                    return carry2

                lax.fori_loop(0, K, scale_row, 0, unroll=2)
                pltpu.sync_copy(rows_v, acc.at[row_v], add=True)

            return carry

        lax.fori_loop(0, iters, chunk_body, 0)
        plsc.subcore_barrier()
        pltpu.sync_copy(acc.at[pl.ds(sub_lo, sub)],
                        out_hbm.at[c, pl.ds(sub_lo, sub)])
        if tail:
            @pl.when(s == NS - 1)
            def _():
                pltpu.sync_copy(acc.at[pl.ds(NS * sub, tail)],
                                out_hbm.at[c, pl.ds(NS * sub, tail)])

    return spmm(row, col, val, emb, zeros)


def _tc_add(partials, s_prev):
    n, h = s_prev.shape
    bn = 1000

    def body(p_ref, sp_ref, e_ref, s_ref):
        e = p_ref[0] + p_ref[1]
        e_ref[...] = e
        s_ref[...] = sp_ref[...] + e

    return pl.pallas_call(
        body,
        grid=(n // bn,),
        in_specs=[
            pl.BlockSpec((2, bn, h), lambda i: (0, i, 0)),
            pl.BlockSpec((bn, h), lambda i: (i, 0)),
        ],
        out_specs=[pl.BlockSpec((bn, h), lambda i: (i, 0))] * 2,
        out_shape=[jax.ShapeDtypeStruct((n, h), jnp.float32)] * 2,
    )(partials, s_prev)


def _gather_dot_sc(s_emb, user, pos, neg):
    n, h = s_emb.shape
    b = user.shape[0]
    kb = b // NWK
    groups = h // LANES
    mesh = plsc.VectorSubcoreMesh(core_axis_name="c", subcore_axis_name="s")

    @functools.partial(
        pl.kernel,
        out_type=(jax.ShapeDtypeStruct((b,), jnp.float32),
                  jax.ShapeDtypeStruct((b,), jnp.float32)),
        mesh=mesh,
        compiler_params=pltpu.CompilerParams(needs_layout_passes=False),
        scratch_types=[
            pltpu.VMEM((kb,), jnp.int32),
            pltpu.VMEM((kb,), jnp.int32),
            pltpu.VMEM((kb,), jnp.int32),
            pltpu.VMEM((kb, h), jnp.float32),
            pltpu.VMEM((kb, h), jnp.float32),
            pltpu.VMEM((kb, h), jnp.float32),
            pltpu.VMEM((kb,), jnp.float32),
            pltpu.VMEM((kb,), jnp.float32),
            pltpu.SemaphoreType.DMA,
        ],
    )
    def gdot(s_hbm, u_hbm, p_hbm, ng_hbm, dpos_hbm, dneg_hbm,
             ui, pi, ni, ur, pr, nr, dp_v, dn_v, sem):
        c = lax.axis_index("c")
        s = lax.axis_index("s")
        wid = s * NC + c
        base = wid * kb
        pltpu.sync_copy(u_hbm.at[pl.ds(base, kb)], ui)
        pltpu.sync_copy(p_hbm.at[pl.ds(base, kb)], pi)
        pltpu.sync_copy(ng_hbm.at[pl.ds(base, kb)], ni)
        pltpu.async_copy(s_hbm.at[ui], ur, sem).wait()
        pltpu.async_copy(s_hbm.at[pi], pr, sem).wait()
        pltpu.async_copy(s_hbm.at[ni], nr, sem).wait()

        last_lane = lax.iota(jnp.int32, LANES) == (LANES - 1)

        def dot_row(r, carry):
            accp = ur[r, pl.ds(0, LANES)] * pr[r, pl.ds(0, LANES)]
            accn = ur[r, pl.ds(0, LANES)] * nr[r, pl.ds(0, LANES)]
            for g in range(1, groups):
                sl = pl.ds(g * LANES, LANES)
                u = ur[r, sl]
                accp = accp + u * pr[r, sl]
                accn = accn + u * nr[r, sl]
            ridx = jnp.full((LANES,), r, jnp.int32)
            plsc.store_scatter(dp_v, [ridx], plsc.cumsum(accp), mask=last_lane)
            plsc.store_scatter(dn_v, [ridx], plsc.cumsum(accn), mask=last_lane)
            return carry

        lax.fori_loop(0, kb, dot_row, 0, unroll=2)
        pltpu.sync_copy(dp_v, dpos_hbm.at[pl.ds(base, kb)])
        pltpu.sync_copy(dn_v, dneg_hbm.at[pl.ds(base, kb)])

    return gdot(s_emb, user, pos, neg)


def _tc_loss(dpos, dneg, inv_scale):
    def body(dp_ref, dn_ref, o_ref):
        x = (dn_ref[...] - dp_ref[...]) * inv_scale
        sp = jnp.log1p(jnp.exp(-jnp.abs(x))) + jnp.maximum(x, 0.0)
        o_ref[0, 0] = jnp.sum(sp)

    out = pl.pallas_call(
        body,
        out_shape=jax.ShapeDtypeStruct((1, 1), jnp.float32),
        out_specs=pl.BlockSpec(memory_space=pltpu.SMEM),
    )(dpos, dneg)
    return out[0, 0]


def kernel(user, pos, neg, laplacian_indices, laplacian_values, emb_weight):
    row = laplacian_indices[0].astype(jnp.int32)
    col = laplacian_indices[1].astype(jnp.int32)
    val = laplacian_values.astype(jnp.float32)
    user = user.astype(jnp.int32)
    pos = pos.astype(jnp.int32)
    neg = neg.astype(jnp.int32)
    zeros = jnp.zeros_like(emb_weight)

    e = emb_weight
    s_sum = emb_weight
    for _ in range(3):
        partials = _spmm_sc(e, row, col, val, zeros)
        e, s_sum = _tc_add(partials, s_sum)

    dpos, dneg = _gather_dot_sc(s_sum, user, pos, neg)
    n_layers_p1 = 4.0
    return _tc_loss(dpos.reshape(32, -1), dneg.reshape(32, -1),
                    1.0 / (n_layers_p1 * n_layers_p1))
